# baseline (speedup 1.0000x reference)
import jax
import jax.numpy as jnp
from jax.experimental import pallas as pl
from jax.experimental.pallas import tpu as pltpu

_BM = 200
_BM2 = 500


def _xw_kernel(x_ref, w_ref, b_ref, o_ref):
    o_ref[...] = (
        jnp.dot(x_ref[...].astype(jnp.bfloat16), w_ref[...], preferred_element_type=jnp.float32)
        + b_ref[...]
    ).astype(jnp.bfloat16)


def _layer1_kernel(a_ref, h_ref, w_ref, b_ref, o_ref, abf_ref):
    a_bf = a_ref[...].astype(jnp.bfloat16)
    abf_ref[...] = a_bf
    acc = jnp.dot(a_bf, h_ref[...], preferred_element_type=jnp.float32)
    acc = jnp.maximum(acc, 0.0).astype(jnp.bfloat16)
    o_ref[...] = (
        jnp.dot(acc, w_ref[...], preferred_element_type=jnp.float32) + b_ref[...]
    ).astype(jnp.bfloat16)


def _mid_kernel(a_ref, h_ref, w_ref, b_ref, o_ref):
    acc = jnp.dot(a_ref[...], h_ref[...], preferred_element_type=jnp.float32)
    acc = jnp.maximum(acc, 0.0).astype(jnp.bfloat16)
    o_ref[...] = (
        jnp.dot(acc, w_ref[...], preferred_element_type=jnp.float32) + b_ref[...]
    ).astype(jnp.bfloat16)


def _final_kernel(a_ref, h_ref, o_ref):
    o_ref[...] = jnp.dot(a_ref[...], h_ref[...], preferred_element_type=jnp.float32)


def _pick_bm(n, want):
    bm = min(want, n)
    while n % bm or bm % 8:
        bm -= 8 if bm % 8 == 0 else bm % 8
        if bm <= 0:
            return n
    return bm


def kernel(x, adj_t, W1, b1, W2, b2, W3, b3):
    n, d_in = x.shape
    d_hid = W1.shape[1]
    d_out = W3.shape[1]
    bm = _pick_bm(n, _BM)
    grid = (n // bm,)
    bm2 = _pick_bm(n, _BM2)
    grid2 = (n // bm2,)

    b1r = b1.reshape(1, -1)
    b2r = b2.reshape(1, -1)
    b3r = b3.reshape(1, -1)
    w1b = W1.astype(jnp.bfloat16)
    w2b = W2.astype(jnp.bfloat16)
    w3b = W3.astype(jnp.bfloat16)

    p1 = pl.pallas_call(
        _xw_kernel,
        grid=grid,
        in_specs=[
            pl.BlockSpec((bm, d_in), lambda i: (i, 0)),
            pl.BlockSpec((d_in, d_hid), lambda i: (0, 0)),
            pl.BlockSpec((1, d_hid), lambda i: (0, 0)),
        ],
        out_specs=pl.BlockSpec((bm, d_hid), lambda i: (i, 0)),
        out_shape=jax.ShapeDtypeStruct((n, d_hid), jnp.bfloat16),
    )(x, w1b, b1r)

    h2, a_bf = pl.pallas_call(
        _layer1_kernel,
        grid=grid,
        in_specs=[
            pl.BlockSpec((bm, n), lambda i: (i, 0)),
            pl.BlockSpec((n, d_hid), lambda i: (0, 0)),
            pl.BlockSpec((d_hid, d_hid), lambda i: (0, 0)),
            pl.BlockSpec((1, d_hid), lambda i: (0, 0)),
        ],
        out_specs=[
            pl.BlockSpec((bm, d_hid), lambda i: (i, 0)),
            pl.BlockSpec((bm, n), lambda i: (i, 0)),
        ],
        out_shape=[
            jax.ShapeDtypeStruct((n, d_hid), jnp.bfloat16),
            jax.ShapeDtypeStruct((n, n), jnp.bfloat16),
        ],
    )(adj_t, p1, w2b, b2r)

    h3 = pl.pallas_call(
        _mid_kernel,
        grid=grid2,
        in_specs=[
            pl.BlockSpec((bm2, n), lambda i: (i, 0)),
            pl.BlockSpec((n, d_hid), lambda i: (0, 0)),
            pl.BlockSpec((d_hid, d_out), lambda i: (0, 0)),
            pl.BlockSpec((1, d_out), lambda i: (0, 0)),
        ],
        out_specs=pl.BlockSpec((bm2, d_out), lambda i: (i, 0)),
        out_shape=jax.ShapeDtypeStruct((n, d_out), jnp.bfloat16),
    )(a_bf, h2, w3b, b3r)

    out = pl.pallas_call(
        _final_kernel,
        grid=grid2,
        in_specs=[
            pl.BlockSpec((bm2, n), lambda i: (i, 0)),
            pl.BlockSpec((n, d_out), lambda i: (0, 0)),
        ],
        out_specs=pl.BlockSpec((bm2, d_out), lambda i: (i, 0)),
        out_shape=jax.ShapeDtypeStruct((n, d_out), jnp.float32),
    )(a_bf, h3)

    return out


# speedup vs baseline: 1.0809x; 1.0809x over previous
import jax
import jax.numpy as jnp
from jax.experimental import pallas as pl
from jax.experimental.pallas import tpu as pltpu

_BM = 500
_BM2 = 1000


def _xw_kernel(x_ref, w_ref, b_ref, o_ref):
    o_ref[...] = (
        jnp.dot(x_ref[...].astype(jnp.bfloat16), w_ref[...], preferred_element_type=jnp.float32)
        + b_ref[...]
    ).astype(jnp.bfloat16)


def _layer1_kernel(a_ref, h_ref, w_ref, b_ref, o_ref, abf_ref):
    a_bf = a_ref[...].astype(jnp.bfloat16)
    abf_ref[...] = a_bf
    acc = jnp.dot(a_bf, h_ref[...], preferred_element_type=jnp.float32)
    acc = jnp.maximum(acc, 0.0).astype(jnp.bfloat16)
    o_ref[...] = (
        jnp.dot(acc, w_ref[...], preferred_element_type=jnp.float32) + b_ref[...]
    ).astype(jnp.bfloat16)


def _mid_kernel(a_ref, h_ref, w_ref, b_ref, o_ref):
    acc = jnp.dot(a_ref[...], h_ref[...], preferred_element_type=jnp.float32)
    acc = jnp.maximum(acc, 0.0).astype(jnp.bfloat16)
    o_ref[...] = (
        jnp.dot(acc, w_ref[...], preferred_element_type=jnp.float32) + b_ref[...]
    ).astype(jnp.bfloat16)


def _final_kernel(a_ref, h_ref, o_ref):
    o_ref[...] = jnp.dot(a_ref[...], h_ref[...], preferred_element_type=jnp.float32)


def _pick_bm(n, want):
    bm = min(want, n)
    while n % bm or bm % 8:
        bm -= 8 if bm % 8 == 0 else bm % 8
        if bm <= 0:
            return n
    return bm


def kernel(x, adj_t, W1, b1, W2, b2, W3, b3):
    n, d_in = x.shape
    d_hid = W1.shape[1]
    d_out = W3.shape[1]
    bm = _pick_bm(n, _BM)
    grid = (n // bm,)
    bm2 = _pick_bm(n, _BM2)
    grid2 = (n // bm2,)

    b1r = b1.reshape(1, -1)
    b2r = b2.reshape(1, -1)
    b3r = b3.reshape(1, -1)
    w1b = W1.astype(jnp.bfloat16)
    w2b = W2.astype(jnp.bfloat16)
    w3b = W3.astype(jnp.bfloat16)

    p1 = pl.pallas_call(
        _xw_kernel,
        grid=grid,
        in_specs=[
            pl.BlockSpec((bm, d_in), lambda i: (i, 0)),
            pl.BlockSpec((d_in, d_hid), lambda i: (0, 0)),
            pl.BlockSpec((1, d_hid), lambda i: (0, 0)),
        ],
        out_specs=pl.BlockSpec((bm, d_hid), lambda i: (i, 0)),
        out_shape=jax.ShapeDtypeStruct((n, d_hid), jnp.bfloat16),
    )(x, w1b, b1r)

    h2, a_bf = pl.pallas_call(
        _layer1_kernel,
        grid=grid,
        in_specs=[
            pl.BlockSpec((bm, n), lambda i: (i, 0)),
            pl.BlockSpec((n, d_hid), lambda i: (0, 0)),
            pl.BlockSpec((d_hid, d_hid), lambda i: (0, 0)),
            pl.BlockSpec((1, d_hid), lambda i: (0, 0)),
        ],
        out_specs=[
            pl.BlockSpec((bm, d_hid), lambda i: (i, 0)),
            pl.BlockSpec((bm, n), lambda i: (i, 0)),
        ],
        out_shape=[
            jax.ShapeDtypeStruct((n, d_hid), jnp.bfloat16),
            jax.ShapeDtypeStruct((n, n), jnp.bfloat16),
        ],
    )(adj_t, p1, w2b, b2r)

    h3 = pl.pallas_call(
        _mid_kernel,
        grid=grid2,
        in_specs=[
            pl.BlockSpec((bm2, n), lambda i: (i, 0)),
            pl.BlockSpec((n, d_hid), lambda i: (0, 0)),
            pl.BlockSpec((d_hid, d_out), lambda i: (0, 0)),
            pl.BlockSpec((1, d_out), lambda i: (0, 0)),
        ],
        out_specs=pl.BlockSpec((bm2, d_out), lambda i: (i, 0)),
        out_shape=jax.ShapeDtypeStruct((n, d_out), jnp.bfloat16),
    )(a_bf, h2, w3b, b3r)

    out = pl.pallas_call(
        _final_kernel,
        grid=grid2,
        in_specs=[
            pl.BlockSpec((bm2, n), lambda i: (i, 0)),
            pl.BlockSpec((n, d_out), lambda i: (0, 0)),
        ],
        out_specs=pl.BlockSpec((bm2, d_out), lambda i: (i, 0)),
        out_shape=jax.ShapeDtypeStruct((n, d_out), jnp.float32),
    )(a_bf, h3)

    return out


# speedup vs baseline: 1.1531x; 1.0667x over previous
import functools as _ft

import jax
import jax.numpy as jnp
from jax.experimental import pallas as pl
from jax.experimental.pallas import tpu as pltpu

_BM = 400
_BM2 = 1000


def _l1_kernel(x_ref, a_ref, w1_ref, b1_ref, w2_ref, b2_ref, h2_ref, abf_ref, p1_ref):
    @pl.when(pl.program_id(0) == 0)
    def _():
        p1_ref[...] = (
            jnp.dot(
                x_ref[...].astype(jnp.bfloat16), w1_ref[...],
                preferred_element_type=jnp.float32,
            )
            + b1_ref[...]
        ).astype(jnp.bfloat16)

    a_bf = a_ref[...].astype(jnp.bfloat16)
    abf_ref[...] = a_bf
    acc = jnp.dot(a_bf, p1_ref[...], preferred_element_type=jnp.float32)
    acc = jnp.maximum(acc, 0.0).astype(jnp.bfloat16)
    h2_ref[...] = (
        jnp.dot(acc, w2_ref[...], preferred_element_type=jnp.float32) + b2_ref[...]
    ).astype(jnp.bfloat16)


def _l23_kernel(nblk, bm2, a_ref, h2_ref, w3_ref, b3_ref, o_ref, h3_ref):
    i = pl.program_id(0)

    @pl.when(i < nblk)
    def _():
        acc = jnp.dot(a_ref[...], h2_ref[...], preferred_element_type=jnp.float32)
        acc = jnp.maximum(acc, 0.0).astype(jnp.bfloat16)
        val = (
            jnp.dot(acc, w3_ref[...], preferred_element_type=jnp.float32) + b3_ref[...]
        ).astype(jnp.bfloat16)
        h3_ref[pl.ds(i * bm2, bm2), :] = val

    @pl.when(i >= nblk)
    def _():
        o_ref[...] = jnp.dot(a_ref[...], h3_ref[...], preferred_element_type=jnp.float32)


def _pick_bm(n, want):
    for bm in range(min(want, n), 0, -1):
        if n % bm == 0 and bm % 8 == 0:
            return bm
    return n


def kernel(x, adj_t, W1, b1, W2, b2, W3, b3):
    n, d_in = x.shape
    d_hid = W1.shape[1]
    d_out = W3.shape[1]
    bm = _pick_bm(n, _BM)
    bm2 = _pick_bm(n, _BM2)
    nblk2 = n // bm2

    b1r = b1.reshape(1, -1)
    b2r = b2.reshape(1, -1)
    b3r = b3.reshape(1, -1)
    w1b = W1.astype(jnp.bfloat16)
    w2b = W2.astype(jnp.bfloat16)
    w3b = W3.astype(jnp.bfloat16)

    h2, a_bf = pl.pallas_call(
        _l1_kernel,
        grid=(n // bm,),
        in_specs=[
            pl.BlockSpec((n, d_in), lambda i: (0, 0)),
            pl.BlockSpec((bm, n), lambda i: (i, 0)),
            pl.BlockSpec((d_in, d_hid), lambda i: (0, 0)),
            pl.BlockSpec((1, d_hid), lambda i: (0, 0)),
            pl.BlockSpec((d_hid, d_hid), lambda i: (0, 0)),
            pl.BlockSpec((1, d_hid), lambda i: (0, 0)),
        ],
        out_specs=[
            pl.BlockSpec((bm, d_hid), lambda i: (i, 0)),
            pl.BlockSpec((bm, n), lambda i: (i, 0)),
        ],
        out_shape=[
            jax.ShapeDtypeStruct((n, d_hid), jnp.bfloat16),
            jax.ShapeDtypeStruct((n, n), jnp.bfloat16),
        ],
        scratch_shapes=[pltpu.VMEM((n, d_hid), jnp.bfloat16)],
    )(x, adj_t, w1b, b1r, w2b, b2r)

    out = pl.pallas_call(
        _ft.partial(_l23_kernel, nblk2, bm2),
        grid=(2 * nblk2,),
        in_specs=[
            pl.BlockSpec((bm2, n), lambda i: (jax.lax.rem(i, nblk2), 0)),
            pl.BlockSpec((n, d_hid), lambda i: (0, 0)),
            pl.BlockSpec((d_hid, d_out), lambda i: (0, 0)),
            pl.BlockSpec((1, d_out), lambda i: (0, 0)),
        ],
        out_specs=pl.BlockSpec((bm2, d_out), lambda i: (jax.lax.rem(i, nblk2), 0)),
        out_shape=jax.ShapeDtypeStruct((n, d_out), jnp.float32),
        scratch_shapes=[pltpu.VMEM((n, d_out), jnp.bfloat16)],
    )(a_bf, h2, w3b, b3r)

    return out
